# Initial kernel scaffold; baseline (speedup 1.0000x reference)
import jax
import jax.numpy as jnp
from jax.experimental import pallas as pl


def kernel(x, p4a_w_b1, p4a_bn_b1_g, p4a_bn_b1_b, p4a_bn_b1_m, p4a_bn_b1_v, p4a_w_2a, p4a_bn_2a_g, p4a_bn_2a_b, p4a_bn_2a_m, p4a_bn_2a_v, p4a_w_2b, p4a_bn_2b_g, p4a_bn_2b_b, p4a_bn_2b_m, p4a_bn_2b_v, p4a_w_2c, p4a_bn_2c_g, p4a_bn_2c_b, p4a_bn_2c_m, p4a_bn_2c_v, p4b_w_2a, p4b_bn_2a_g, p4b_bn_2a_b, p4b_bn_2a_m, p4b_bn_2a_v, p4b_w_2b, p4b_bn_2b_g, p4b_bn_2b_b, p4b_bn_2b_m, p4b_bn_2b_v, p4b_w_2c, p4b_bn_2c_g, p4b_bn_2c_b, p4b_bn_2c_m, p4b_bn_2c_v, p4c_w_2a, p4c_bn_2a_g, p4c_bn_2a_b, p4c_bn_2a_m, p4c_bn_2a_v, p4c_w_2b, p4c_bn_2b_g, p4c_bn_2b_b, p4c_bn_2b_m, p4c_bn_2b_v, p4c_w_2c, p4c_bn_2c_g, p4c_bn_2c_b, p4c_bn_2c_m, p4c_bn_2c_v, p4d_w_2a, p4d_bn_2a_g, p4d_bn_2a_b, p4d_bn_2a_m, p4d_bn_2a_v, p4d_w_2b, p4d_bn_2b_g, p4d_bn_2b_b, p4d_bn_2b_m, p4d_bn_2b_v, p4d_w_2c, p4d_bn_2c_g, p4d_bn_2c_b, p4d_bn_2c_m, p4d_bn_2c_v, p4e_w_2a, p4e_bn_2a_g, p4e_bn_2a_b, p4e_bn_2a_m, p4e_bn_2a_v, p4e_w_2b, p4e_bn_2b_g, p4e_bn_2b_b, p4e_bn_2b_m, p4e_bn_2b_v, p4e_w_2c, p4e_bn_2c_g, p4e_bn_2c_b, p4e_bn_2c_m, p4e_bn_2c_v, p4f_w_2a, p4f_bn_2a_g, p4f_bn_2a_b, p4f_bn_2a_m, p4f_bn_2a_v, p4f_w_2b, p4f_bn_2b_g, p4f_bn_2b_b, p4f_bn_2b_m, p4f_bn_2b_v, p4f_w_2c, p4f_bn_2c_g, p4f_bn_2c_b, p4f_bn_2c_m, p4f_bn_2c_v):
    raise NotImplementedError("write your pallas kernel here")


# speedup vs baseline: 2.6896x; 2.6896x over previous
import jax
import jax.numpy as jnp
from jax.experimental import pallas as pl
from jax.experimental.pallas import tpu as pltpu

_BN_EPS = 9.999999747378752e-06

_N = 16
_HW = 14
_WP = 16
_M = _HW * _WP
_LP = 264


def _fold(g_ref, b_ref, m_ref, v_ref):
    g, b, m, v = g_ref[...], b_ref[...], m_ref[...], v_ref[...]
    s = g * jax.lax.rsqrt(v + _BN_EPS)
    return s, b - m * s


def _layer4_kernel(*args):
    it = iter(args)
    x_ref = next(it)
    wb1 = next(it)
    bnb1 = [next(it) for _ in range(4)]
    w2a_a = next(it)
    bn2a_a = [next(it) for _ in range(4)]
    w2c_a = next(it)
    bn2c_a = [next(it) for _ in range(4)]
    w2b_all = next(it)
    bn2b = [[next(it) for _ in range(4)] for _ in range(6)]
    ident = []
    for _ in range(5):
        w2a = next(it)
        bn2a = [next(it) for _ in range(4)]
        w2c = next(it)
        bn2c = [next(it) for _ in range(4)]
        ident.append((w2a, bn2a, w2c, bn2c))
    o_ref = next(it)
    xp = next(it)

    mask = (jax.lax.broadcasted_iota(jnp.int32, (_M, 1), 0) % _WP
            < _HW).astype(jnp.float32)

    def mm(xv, w_ref, s, sh, relu, extra=None):
        acc = jax.lax.dot_general(
            xv, w_ref[...], (((1,), (1,)), ((), ())),
            preferred_element_type=jnp.float32)
        acc = acc * s + sh
        if extra is not None:
            acc = acc + extra
        return jnp.maximum(acc, 0.0) if relu else acc

    def conv3(y1, blk, s, sh):
        xp[17:17 + _M, :] = y1 * mask
        acc = jnp.zeros((_M, 256), jnp.float32)
        for dy in range(3):
            for dx in range(3):
                off = dy * _WP + dx
                acc = acc + jnp.dot(xp[off:off + _M, :],
                                    w2b_all[blk, dy * 3 + dx],
                                    preferred_element_type=jnp.float32)
        return jnp.maximum(acc * s + sh, 0.0)

    xp[0:17, :] = jnp.zeros((17, 256), jnp.float32)
    xp[17 + _M:_LP, :] = jnp.zeros((_LP - 17 - _M, 256), jnp.float32)

    x = x_ref[0]

    s, sh = _fold(*bnb1)
    b1 = mm(x, wb1, s, sh, relu=False)
    s, sh = _fold(*bn2a_a)
    y1 = mm(x, w2a_a, s, sh, relu=True)
    s, sh = _fold(*bn2b[0])
    y2 = conv3(y1, 0, s, sh)
    s, sh = _fold(*bn2c_a)
    o_ref[0] = mm(y2, w2c_a, s, sh, relu=True, extra=b1)

    for i, (w2a, bn2a, w2c, bn2c) in enumerate(ident):
        prev = o_ref[0]
        s, sh = _fold(*bn2a)
        y1 = mm(prev, w2a, s, sh, relu=True)
        s, sh = _fold(*bn2b[i + 1])
        y2 = conv3(y1, i + 1, s, sh)
        s, sh = _fold(*bn2c)
        o_ref[0] = mm(y2, w2c, s, sh, relu=True, extra=prev)


def kernel(x, p4a_w_b1, p4a_bn_b1_g, p4a_bn_b1_b, p4a_bn_b1_m, p4a_bn_b1_v, p4a_w_2a, p4a_bn_2a_g, p4a_bn_2a_b, p4a_bn_2a_m, p4a_bn_2a_v, p4a_w_2b, p4a_bn_2b_g, p4a_bn_2b_b, p4a_bn_2b_m, p4a_bn_2b_v, p4a_w_2c, p4a_bn_2c_g, p4a_bn_2c_b, p4a_bn_2c_m, p4a_bn_2c_v, p4b_w_2a, p4b_bn_2a_g, p4b_bn_2a_b, p4b_bn_2a_m, p4b_bn_2a_v, p4b_w_2b, p4b_bn_2b_g, p4b_bn_2b_b, p4b_bn_2b_m, p4b_bn_2b_v, p4b_w_2c, p4b_bn_2c_g, p4b_bn_2c_b, p4b_bn_2c_m, p4b_bn_2c_v, p4c_w_2a, p4c_bn_2a_g, p4c_bn_2a_b, p4c_bn_2a_m, p4c_bn_2a_v, p4c_w_2b, p4c_bn_2b_g, p4c_bn_2b_b, p4c_bn_2b_m, p4c_bn_2b_v, p4c_w_2c, p4c_bn_2c_g, p4c_bn_2c_b, p4c_bn_2c_m, p4c_bn_2c_v, p4d_w_2a, p4d_bn_2a_g, p4d_bn_2a_b, p4d_bn_2a_m, p4d_bn_2a_v, p4d_w_2b, p4d_bn_2b_g, p4d_bn_2b_b, p4d_bn_2b_m, p4d_bn_2b_v, p4d_w_2c, p4d_bn_2c_g, p4d_bn_2c_b, p4d_bn_2c_m, p4d_bn_2c_v, p4e_w_2a, p4e_bn_2a_g, p4e_bn_2a_b, p4e_bn_2a_m, p4e_bn_2a_v, p4e_w_2b, p4e_bn_2b_g, p4e_bn_2b_b, p4e_bn_2b_m, p4e_bn_2b_v, p4e_w_2c, p4e_bn_2c_g, p4e_bn_2c_b, p4e_bn_2c_m, p4e_bn_2c_v, p4f_w_2a, p4f_bn_2a_g, p4f_bn_2a_b, p4f_bn_2a_m, p4f_bn_2a_v, p4f_w_2b, p4f_bn_2b_g, p4f_bn_2b_b, p4f_bn_2b_m, p4f_bn_2b_v, p4f_w_2c, p4f_bn_2c_g, p4f_bn_2c_b, p4f_bn_2c_m, p4f_bn_2c_v):
    xs = jnp.transpose(x[:, :, ::2, ::2], (0, 2, 3, 1)).astype(jnp.float32)
    xs = jnp.pad(xs, ((0, 0), (0, 0), (0, 2), (0, 0))).reshape(_N, _M, 512)

    w2b_all = jnp.transpose(
        jnp.stack([p4a_w_2b, p4b_w_2b, p4c_w_2b, p4d_w_2b, p4e_w_2b, p4f_w_2b]),
        (0, 3, 4, 2, 1)).reshape(6, 9, 256, 256)

    r2 = lambda w: w.reshape(w.shape[0], w.shape[1])
    rv = lambda t: t.reshape(1, -1)

    operands = [xs,
                r2(p4a_w_b1), rv(p4a_bn_b1_g), rv(p4a_bn_b1_b), rv(p4a_bn_b1_m), rv(p4a_bn_b1_v),
                r2(p4a_w_2a), rv(p4a_bn_2a_g), rv(p4a_bn_2a_b), rv(p4a_bn_2a_m), rv(p4a_bn_2a_v),
                r2(p4a_w_2c), rv(p4a_bn_2c_g), rv(p4a_bn_2c_b), rv(p4a_bn_2c_m), rv(p4a_bn_2c_v),
                w2b_all,
                rv(p4a_bn_2b_g), rv(p4a_bn_2b_b), rv(p4a_bn_2b_m), rv(p4a_bn_2b_v),
                rv(p4b_bn_2b_g), rv(p4b_bn_2b_b), rv(p4b_bn_2b_m), rv(p4b_bn_2b_v),
                rv(p4c_bn_2b_g), rv(p4c_bn_2b_b), rv(p4c_bn_2b_m), rv(p4c_bn_2b_v),
                rv(p4d_bn_2b_g), rv(p4d_bn_2b_b), rv(p4d_bn_2b_m), rv(p4d_bn_2b_v),
                rv(p4e_bn_2b_g), rv(p4e_bn_2b_b), rv(p4e_bn_2b_m), rv(p4e_bn_2b_v),
                rv(p4f_bn_2b_g), rv(p4f_bn_2b_b), rv(p4f_bn_2b_m), rv(p4f_bn_2b_v)]
    for wa, ga, ba, ma, va, wc, gc, bc, mc, vc in (
        (p4b_w_2a, p4b_bn_2a_g, p4b_bn_2a_b, p4b_bn_2a_m, p4b_bn_2a_v,
         p4b_w_2c, p4b_bn_2c_g, p4b_bn_2c_b, p4b_bn_2c_m, p4b_bn_2c_v),
        (p4c_w_2a, p4c_bn_2a_g, p4c_bn_2a_b, p4c_bn_2a_m, p4c_bn_2a_v,
         p4c_w_2c, p4c_bn_2c_g, p4c_bn_2c_b, p4c_bn_2c_m, p4c_bn_2c_v),
        (p4d_w_2a, p4d_bn_2a_g, p4d_bn_2a_b, p4d_bn_2a_m, p4d_bn_2a_v,
         p4d_w_2c, p4d_bn_2c_g, p4d_bn_2c_b, p4d_bn_2c_m, p4d_bn_2c_v),
        (p4e_w_2a, p4e_bn_2a_g, p4e_bn_2a_b, p4e_bn_2a_m, p4e_bn_2a_v,
         p4e_w_2c, p4e_bn_2c_g, p4e_bn_2c_b, p4e_bn_2c_m, p4e_bn_2c_v),
        (p4f_w_2a, p4f_bn_2a_g, p4f_bn_2a_b, p4f_bn_2a_m, p4f_bn_2a_v,
         p4f_w_2c, p4f_bn_2c_g, p4f_bn_2c_b, p4f_bn_2c_m, p4f_bn_2c_v)):
        operands += [r2(wa), rv(ga), rv(ba), rv(ma), rv(va),
                     r2(wc), rv(gc), rv(bc), rv(mc), rv(vc)]

    def spec(a):
        if a.ndim == 3:
            return pl.BlockSpec((1, _M, 512), lambda n: (n, 0, 0))
        if a.ndim == 4:
            return pl.BlockSpec(a.shape, lambda n: (0, 0, 0, 0))
        return pl.BlockSpec(a.shape, lambda n: (0, 0))

    out = pl.pallas_call(
        _layer4_kernel,
        out_shape=jax.ShapeDtypeStruct((_N, _M, 1024), jnp.float32),
        grid_spec=pltpu.PrefetchScalarGridSpec(
            num_scalar_prefetch=0,
            grid=(_N,),
            in_specs=[spec(a) for a in operands],
            out_specs=pl.BlockSpec((1, _M, 1024), lambda n: (n, 0, 0)),
            scratch_shapes=[pltpu.VMEM((_LP, 256), jnp.float32)],
        ),
        compiler_params=pltpu.CompilerParams(
            dimension_semantics=("parallel",),
            vmem_limit_bytes=100 * 1024 * 1024,
        ),
    )(*operands)

    return jnp.transpose(out.reshape(_N, _HW, _WP, 1024)[:, :, :_HW, :],
                         (0, 3, 1, 2))


# speedup vs baseline: 2.7618x; 1.0269x over previous
import jax
import jax.numpy as jnp
from jax.experimental import pallas as pl
from jax.experimental.pallas import tpu as pltpu

_BN_EPS = 9.999999747378752e-06

_N = 16
_HW = 14
_WP = 16
_M = _HW * _WP
_LP = 264


def _fold(g_ref, b_ref, m_ref, v_ref):
    g, b, m, v = g_ref[...], b_ref[...], m_ref[...], v_ref[...]
    return b - m * (g * jax.lax.rsqrt(v + _BN_EPS))


def _layer4_kernel(*args):
    it = iter(args)
    x_ref = next(it)
    wb1 = next(it)
    bnb1 = [next(it) for _ in range(4)]
    w2a_a = next(it)
    bn2a_a = [next(it) for _ in range(4)]
    w2c_a = next(it)
    bn2c_a = [next(it) for _ in range(4)]
    w2b_all = next(it)
    bn2b = [[next(it) for _ in range(4)] for _ in range(6)]
    ident = []
    for _ in range(5):
        w2a = next(it)
        bn2a = [next(it) for _ in range(4)]
        w2c = next(it)
        bn2c = [next(it) for _ in range(4)]
        ident.append((w2a, bn2a, w2c, bn2c))
    o_ref = next(it)
    xp = next(it)

    mask = (jax.lax.broadcasted_iota(jnp.int32, (_M, 1), 0) % _WP
            < _HW).astype(jnp.float32)

    def mm(xv, w_ref, sh, relu, extra=None):
        acc = jax.lax.dot_general(
            xv, w_ref[...], (((1,), (1,)), ((), ())),
            preferred_element_type=jnp.float32)
        acc = acc + sh
        if extra is not None:
            acc = acc + extra
        return jnp.maximum(acc, 0.0) if relu else acc

    def conv3(y1, blk, sh):
        xp[17:17 + _M, :] = (y1 * mask).astype(jnp.bfloat16)
        acc = sh + jnp.zeros((_M, 256), jnp.float32)
        for dy in range(3):
            for dx in range(3):
                off = dy * _WP + dx
                acc = acc + jnp.dot(xp[off:off + _M, :],
                                    w2b_all[blk, dy * 3 + dx],
                                    preferred_element_type=jnp.float32)
        return jnp.maximum(acc, 0.0)

    xp[0:17, :] = jnp.zeros((17, 256), jnp.bfloat16)
    xp[17 + _M:_LP, :] = jnp.zeros((_LP - 17 - _M, 256), jnp.bfloat16)

    x = x_ref[0]

    b1 = mm(x, wb1, _fold(*bnb1), relu=False)
    y1 = mm(x, w2a_a, _fold(*bn2a_a), relu=True)
    y2 = conv3(y1, 0, _fold(*bn2b[0]))
    o_ref[0] = mm(y2.astype(jnp.bfloat16), w2c_a, _fold(*bn2c_a),
                  relu=True, extra=b1)

    for i, (w2a, bn2a, w2c, bn2c) in enumerate(ident):
        prev = o_ref[0]
        y1 = mm(prev.astype(jnp.bfloat16), w2a, _fold(*bn2a), relu=True)
        y2 = conv3(y1, i + 1, _fold(*bn2b[i + 1]))
        o_ref[0] = mm(y2.astype(jnp.bfloat16), w2c, _fold(*bn2c),
                      relu=True, extra=prev)


def kernel(x, p4a_w_b1, p4a_bn_b1_g, p4a_bn_b1_b, p4a_bn_b1_m, p4a_bn_b1_v, p4a_w_2a, p4a_bn_2a_g, p4a_bn_2a_b, p4a_bn_2a_m, p4a_bn_2a_v, p4a_w_2b, p4a_bn_2b_g, p4a_bn_2b_b, p4a_bn_2b_m, p4a_bn_2b_v, p4a_w_2c, p4a_bn_2c_g, p4a_bn_2c_b, p4a_bn_2c_m, p4a_bn_2c_v, p4b_w_2a, p4b_bn_2a_g, p4b_bn_2a_b, p4b_bn_2a_m, p4b_bn_2a_v, p4b_w_2b, p4b_bn_2b_g, p4b_bn_2b_b, p4b_bn_2b_m, p4b_bn_2b_v, p4b_w_2c, p4b_bn_2c_g, p4b_bn_2c_b, p4b_bn_2c_m, p4b_bn_2c_v, p4c_w_2a, p4c_bn_2a_g, p4c_bn_2a_b, p4c_bn_2a_m, p4c_bn_2a_v, p4c_w_2b, p4c_bn_2b_g, p4c_bn_2b_b, p4c_bn_2b_m, p4c_bn_2b_v, p4c_w_2c, p4c_bn_2c_g, p4c_bn_2c_b, p4c_bn_2c_m, p4c_bn_2c_v, p4d_w_2a, p4d_bn_2a_g, p4d_bn_2a_b, p4d_bn_2a_m, p4d_bn_2a_v, p4d_w_2b, p4d_bn_2b_g, p4d_bn_2b_b, p4d_bn_2b_m, p4d_bn_2b_v, p4d_w_2c, p4d_bn_2c_g, p4d_bn_2c_b, p4d_bn_2c_m, p4d_bn_2c_v, p4e_w_2a, p4e_bn_2a_g, p4e_bn_2a_b, p4e_bn_2a_m, p4e_bn_2a_v, p4e_w_2b, p4e_bn_2b_g, p4e_bn_2b_b, p4e_bn_2b_m, p4e_bn_2b_v, p4e_w_2c, p4e_bn_2c_g, p4e_bn_2c_b, p4e_bn_2c_m, p4e_bn_2c_v, p4f_w_2a, p4f_bn_2a_g, p4f_bn_2a_b, p4f_bn_2a_m, p4f_bn_2a_v, p4f_w_2b, p4f_bn_2b_g, p4f_bn_2b_b, p4f_bn_2b_m, p4f_bn_2b_v, p4f_w_2c, p4f_bn_2c_g, p4f_bn_2c_b, p4f_bn_2c_m, p4f_bn_2c_v):
    xs = jnp.transpose(x[:, :, ::2, ::2], (0, 2, 3, 1))
    xs = jnp.pad(xs, ((0, 0), (0, 0), (0, 2), (0, 0))) \
            .reshape(_N, _M, 512).astype(jnp.bfloat16)

    def scale(g, v):
        return g / jnp.sqrt(v + _BN_EPS)

    def w1x1(w, g, v):
        s = scale(g, v)
        return (w.reshape(w.shape[0], w.shape[1]) * s[:, None]).astype(jnp.bfloat16)

    s2b = jnp.stack([scale(p4a_bn_2b_g, p4a_bn_2b_v), scale(p4b_bn_2b_g, p4b_bn_2b_v),
                     scale(p4c_bn_2b_g, p4c_bn_2b_v), scale(p4d_bn_2b_g, p4d_bn_2b_v),
                     scale(p4e_bn_2b_g, p4e_bn_2b_v), scale(p4f_bn_2b_g, p4f_bn_2b_v)])
    w2b_all = jnp.stack([p4a_w_2b, p4b_w_2b, p4c_w_2b, p4d_w_2b, p4e_w_2b, p4f_w_2b])
    w2b_all = jnp.transpose(w2b_all * s2b[:, :, None, None, None],
                            (0, 3, 4, 2, 1)).reshape(6, 9, 256, 256).astype(jnp.bfloat16)

    rv = lambda t: t.reshape(1, -1)

    operands = [xs,
                w1x1(p4a_w_b1, p4a_bn_b1_g, p4a_bn_b1_v),
                rv(p4a_bn_b1_g), rv(p4a_bn_b1_b), rv(p4a_bn_b1_m), rv(p4a_bn_b1_v),
                w1x1(p4a_w_2a, p4a_bn_2a_g, p4a_bn_2a_v),
                rv(p4a_bn_2a_g), rv(p4a_bn_2a_b), rv(p4a_bn_2a_m), rv(p4a_bn_2a_v),
                w1x1(p4a_w_2c, p4a_bn_2c_g, p4a_bn_2c_v),
                rv(p4a_bn_2c_g), rv(p4a_bn_2c_b), rv(p4a_bn_2c_m), rv(p4a_bn_2c_v),
                w2b_all,
                rv(p4a_bn_2b_g), rv(p4a_bn_2b_b), rv(p4a_bn_2b_m), rv(p4a_bn_2b_v),
                rv(p4b_bn_2b_g), rv(p4b_bn_2b_b), rv(p4b_bn_2b_m), rv(p4b_bn_2b_v),
                rv(p4c_bn_2b_g), rv(p4c_bn_2b_b), rv(p4c_bn_2b_m), rv(p4c_bn_2b_v),
                rv(p4d_bn_2b_g), rv(p4d_bn_2b_b), rv(p4d_bn_2b_m), rv(p4d_bn_2b_v),
                rv(p4e_bn_2b_g), rv(p4e_bn_2b_b), rv(p4e_bn_2b_m), rv(p4e_bn_2b_v),
                rv(p4f_bn_2b_g), rv(p4f_bn_2b_b), rv(p4f_bn_2b_m), rv(p4f_bn_2b_v)]
    for wa, ga, ba, ma, va, wc, gc, bc, mc, vc in (
        (p4b_w_2a, p4b_bn_2a_g, p4b_bn_2a_b, p4b_bn_2a_m, p4b_bn_2a_v,
         p4b_w_2c, p4b_bn_2c_g, p4b_bn_2c_b, p4b_bn_2c_m, p4b_bn_2c_v),
        (p4c_w_2a, p4c_bn_2a_g, p4c_bn_2a_b, p4c_bn_2a_m, p4c_bn_2a_v,
         p4c_w_2c, p4c_bn_2c_g, p4c_bn_2c_b, p4c_bn_2c_m, p4c_bn_2c_v),
        (p4d_w_2a, p4d_bn_2a_g, p4d_bn_2a_b, p4d_bn_2a_m, p4d_bn_2a_v,
         p4d_w_2c, p4d_bn_2c_g, p4d_bn_2c_b, p4d_bn_2c_m, p4d_bn_2c_v),
        (p4e_w_2a, p4e_bn_2a_g, p4e_bn_2a_b, p4e_bn_2a_m, p4e_bn_2a_v,
         p4e_w_2c, p4e_bn_2c_g, p4e_bn_2c_b, p4e_bn_2c_m, p4e_bn_2c_v),
        (p4f_w_2a, p4f_bn_2a_g, p4f_bn_2a_b, p4f_bn_2a_m, p4f_bn_2a_v,
         p4f_w_2c, p4f_bn_2c_g, p4f_bn_2c_b, p4f_bn_2c_m, p4f_bn_2c_v)):
        operands += [w1x1(wa, ga, va), rv(ga), rv(ba), rv(ma), rv(va),
                     w1x1(wc, gc, vc), rv(gc), rv(bc), rv(mc), rv(vc)]

    def spec(a):
        if a.ndim == 3:
            return pl.BlockSpec((1, _M, 512), lambda n: (n, 0, 0))
        if a.ndim == 4:
            return pl.BlockSpec(a.shape, lambda n: (0, 0, 0, 0))
        return pl.BlockSpec(a.shape, lambda n: (0, 0))

    out = pl.pallas_call(
        _layer4_kernel,
        out_shape=jax.ShapeDtypeStruct((_N, _M, 1024), jnp.float32),
        grid_spec=pltpu.PrefetchScalarGridSpec(
            num_scalar_prefetch=0,
            grid=(_N,),
            in_specs=[spec(a) for a in operands],
            out_specs=pl.BlockSpec((1, _M, 1024), lambda n: (n, 0, 0)),
            scratch_shapes=[pltpu.VMEM((_LP, 256), jnp.bfloat16)],
        ),
        compiler_params=pltpu.CompilerParams(
            dimension_semantics=("parallel",),
            vmem_limit_bytes=100 * 1024 * 1024,
        ),
    )(*operands)

    return jnp.transpose(out.reshape(_N, _HW, _WP, 1024)[:, :, :_HW, :],
                         (0, 3, 1, 2))
